# baseline (speedup 1.0000x reference)
import functools

import jax
import jax.numpy as jnp
from jax import lax
from jax.experimental import pallas as pl
from jax.experimental.pallas import tpu as pltpu
from jax.experimental.pallas import tpu_sc as plsc

_CHUNK = 128


@functools.lru_cache(maxsize=None)
def _make_gather(batch: int, embed_dim: int):
    info = plsc.get_sparse_core_info()
    nc, ns = info.num_cores, info.num_subcores
    nw = nc * ns
    b_per_w = batch // nw
    n_chunk = b_per_w // _CHUNK
    mesh = plsc.VectorSubcoreMesh(core_axis_name="c", subcore_axis_name="s")

    @functools.partial(
        pl.kernel,
        mesh=mesh,
        out_type=jax.ShapeDtypeStruct((batch, embed_dim), jnp.float32),
        scratch_types=[
            pltpu.VMEM((n_chunk, _CHUNK), jnp.int32),
            pltpu.VMEM((b_per_w, embed_dim), jnp.float32),
            pltpu.SemaphoreType.DMA,
        ],
        compiler_params=pltpu.CompilerParams(use_tc_tiling_on_sc=False),
    )
    def gather_kernel(idx_hbm, table_hbm, out_hbm, idx_v, rows_v, sem):
        wid = lax.axis_index("s") * nc + lax.axis_index("c")
        pltpu.sync_copy(idx_hbm.at[pl.ds(wid * n_chunk, n_chunk)], idx_v)
        copies = []
        for j in range(n_chunk):
            copies.append(
                pltpu.async_copy(
                    table_hbm.at[idx_v.at[j]],
                    rows_v.at[pl.ds(j * _CHUNK, _CHUNK)],
                    sem,
                )
            )
        for c in copies:
            c.wait()
        pltpu.sync_copy(rows_v, out_hbm.at[pl.ds(wid * b_per_w, b_per_w)])

    return gather_kernel


def kernel(task_ids, embedding_weight):
    if task_ids.ndim == 2:
        task_ids = task_ids[:, 0]
    batch = task_ids.shape[0]
    embed_dim = embedding_weight.shape[1]
    idx2d = task_ids.astype(jnp.int32).reshape(batch // _CHUNK, _CHUNK)
    fn = _make_gather(batch, embed_dim)
    return fn(idx2d, embedding_weight)


# speedup vs baseline: 1.4775x; 1.4775x over previous
import functools

import jax
import jax.numpy as jnp
from jax import lax
from jax.experimental import pallas as pl
from jax.experimental.pallas import tpu as pltpu
from jax.experimental.pallas import tpu_sc as plsc


@functools.lru_cache(maxsize=None)
def _make_gather(batch: int, embed_dim: int):
    info = plsc.get_sparse_core_info()
    nc, ns = info.num_cores, info.num_subcores
    nw = nc * ns
    b_per_w = batch // nw
    mesh = plsc.VectorSubcoreMesh(core_axis_name="c", subcore_axis_name="s")

    @functools.partial(
        pl.kernel,
        mesh=mesh,
        out_type=jax.ShapeDtypeStruct((batch, embed_dim), jnp.float32),
        scratch_types=[
            pltpu.VMEM((b_per_w,), jnp.int32),
            pltpu.VMEM((b_per_w, embed_dim), jnp.float32),
            pltpu.SemaphoreType.DMA,
        ],
    )
    def gather_kernel(idx_hbm, table_hbm, out_hbm, idx_v, rows_v, sem):
        wid = lax.axis_index("s") * nc + lax.axis_index("c")
        base = wid * b_per_w
        pltpu.sync_copy(idx_hbm.at[pl.ds(base, b_per_w)], idx_v)

        lanes = 16

        def body(g, _):
            ids = idx_v[pl.ds(g * lanes, lanes)]
            for j in range(lanes):
                pltpu.make_async_copy(
                    table_hbm.at[pl.ds(ids[j], 1)],
                    rows_v.at[pl.ds(g * lanes + j, 1)],
                    sem,
                ).start()
            return 0

        lax.fori_loop(0, b_per_w // lanes, body, 0)
        pltpu.make_async_copy(
            table_hbm.at[pl.ds(0, b_per_w)],
            rows_v,
            sem,
        ).wait()
        pltpu.sync_copy(rows_v, out_hbm.at[pl.ds(base, b_per_w)])

    return gather_kernel


def kernel(task_ids, embedding_weight):
    if task_ids.ndim == 2:
        task_ids = task_ids[:, 0]
    batch = task_ids.shape[0]
    embed_dim = embedding_weight.shape[1]
    fn = _make_gather(batch, embed_dim)
    return fn(task_ids.astype(jnp.int32), embedding_weight)


# speedup vs baseline: 2.1645x; 1.4650x over previous
import functools

import jax
import jax.numpy as jnp
from jax import lax
from jax.experimental import pallas as pl
from jax.experimental.pallas import tpu as pltpu
from jax.experimental.pallas import tpu_sc as plsc

_LANES = 16
_OUT_CHUNK = 2048


@functools.lru_cache(maxsize=None)
def _make_gather(batch: int, embed_dim: int, num_tasks: int):
    info = plsc.get_sparse_core_info()
    nc, ns = info.num_cores, info.num_subcores
    nw = nc * ns
    dims_per_w = embed_dim // nw
    n_chunk = batch // _OUT_CHUNK
    unroll = 8
    mesh = plsc.VectorSubcoreMesh(core_axis_name="c", subcore_axis_name="s")

    @functools.partial(
        pl.kernel,
        mesh=mesh,
        out_type=jax.ShapeDtypeStruct((embed_dim, batch), jnp.float32),
        scratch_types=[
            pltpu.VMEM((batch,), jnp.int32),
            pltpu.VMEM((num_tasks,), jnp.float32),
            pltpu.VMEM((_OUT_CHUNK,), jnp.float32),
        ],
        compiler_params=pltpu.CompilerParams(needs_layout_passes=False),
    )
    def gather_kernel(idx_hbm, tablet_hbm, outt_hbm, ids_v, row_v, out_v):
        wid = lax.axis_index("s") * nc + lax.axis_index("c")
        pltpu.sync_copy(idx_hbm, ids_v)
        for j in range(dims_per_w):
            d = wid * dims_per_w + j
            pltpu.sync_copy(tablet_hbm.at[d], row_v)
            for c in range(n_chunk):

                def body(g, _, c=c):
                    base = g * (_LANES * unroll)
                    for u in range(unroll):
                        o = base + u * _LANES
                        idx16 = ids_v[pl.ds(c * _OUT_CHUNK + o, _LANES)]
                        out_v[pl.ds(o, _LANES)] = plsc.load_gather(
                            row_v, [idx16]
                        )
                    return 0

                lax.fori_loop(0, _OUT_CHUNK // (_LANES * unroll), body, 0)
                pltpu.sync_copy(
                    out_v, outt_hbm.at[d, pl.ds(c * _OUT_CHUNK, _OUT_CHUNK)]
                )

    return gather_kernel


def kernel(task_ids, embedding_weight):
    if task_ids.ndim == 2:
        task_ids = task_ids[:, 0]
    batch = task_ids.shape[0]
    num_tasks, embed_dim = embedding_weight.shape
    fn = _make_gather(batch, embed_dim, num_tasks)
    outt = fn(task_ids.astype(jnp.int32), embedding_weight.T)
    return outt.T
